# baseline (speedup 1.0000x reference)
import functools
import math

import jax
import jax.numpy as jnp
from jax import lax
from jax.experimental import pallas as pl
from jax.experimental.pallas import tpu as pltpu
from jax.experimental.pallas import tpu_sc as plsc

B = 128
N = 32768
K = math.ceil(0.05 * N)

NC = 2
NS = 16
L = 16
NW = NC * NS
ROWS_PER_W = B // NW
NV = N // L
NBKT = 256
HSTRIDE = 257
HWORDS = 264 * L
TW = 128


def _scal(v):
    if getattr(v, "ndim", 0) == 1:
        return jnp.max(v)
    return v


_mesh = plsc.VectorSubcoreMesh(core_axis_name="c", subcore_axis_name="s")


@functools.partial(
    pl.kernel,
    mesh=_mesh,
    out_type=jax.ShapeDtypeStruct((B * TW,), jnp.float32),
    compiler_params=pltpu.CompilerParams(needs_layout_passes=False),
    scratch_types=[
        pltpu.VMEM((N,), jnp.float32),
        pltpu.VMEM((N,), jnp.int32),
        pltpu.VMEM((N,), jnp.int32),
        pltpu.VMEM((HWORDS,), jnp.int32),
        pltpu.VMEM((NBKT,), jnp.int32),
        pltpu.VMEM((L,), jnp.int32),
        pltpu.VMEM((ROWS_PER_W * TW,), jnp.float32),
    ],
)
def _select_thr(x_hbm, thr_hbm, xv, ca, cb, hist, merged, gtot, tbuf):
    wid = lax.axis_index("s") * NC + lax.axis_index("c")
    lanes = lax.iota(jnp.int32, L)
    laneoff = lanes * HSTRIDE
    ones = jnp.ones((L,), jnp.int32)
    zero16 = jnp.zeros((L,), jnp.int32)
    true16 = lanes >= 0

    def clear_hist():
        @plsc.parallel_loop(0, HWORDS // L, unroll=8)
        def _(i):
            hist[pl.ds(i * L, L)] = zero16

    def search(r0):
        @plsc.parallel_loop(0, L, unroll=4)
        def _(g):
            acc = zero16
            for l in range(L):
                acc = acc + hist[pl.ds(l * HSTRIDE + g * L, L)]
            merged[pl.ds(g * L, L)] = acc
            tot = zero16 + jnp.sum(acc)
            plsc.store_scatter(gtot, [zero16 + g], tot, mask=lanes == 0)

        gt = gtot[pl.ds(0, L)]
        grev = lax.rev(gt, (0,))
        gc = plsc.cumsum(grev)
        gmask = gc >= r0
        t0 = _scal(plsc.all_reduce_ffs(gmask))
        gsel = lanes == t0
        gci = jnp.sum(jnp.where(gsel, gc, zero16))
        gri = jnp.sum(jnp.where(gsel, grev, zero16))
        grp = 15 - t0
        rg = r0 - (gci - gri)

        acc = merged[pl.ds(grp * L, L)]
        rev = lax.rev(acc, (0,))
        c = plsc.cumsum(rev)
        mge = c >= rg
        i0 = _scal(plsc.all_reduce_ffs(mge))
        msel = lanes == i0
        ci0 = jnp.sum(jnp.where(msel, c, zero16))
        ri0 = jnp.sum(jnp.where(msel, rev, zero16))
        bucket = grp * L + (15 - i0)
        r_next = rg - (ci0 - ri0)
        return bucket, r_next

    def compact(src, dst, n_src, shift, p):
        clear_hist()
        nv = (n_src + L - 1) // L

        @plsc.parallel_loop(0, nv, unroll=8, carry=zero16)
        def off(i, off):
            sv = src[pl.ds(i * L, L)]
            inb = (i * L + lanes) < n_src
            m = jnp.logical_and((sv >> shift) == p, inb)
            mi = jnp.where(m, ones, zero16)
            pos = off + plsc.cumsum(mi) - 1
            plsc.store_scatter(dst, [pos], sv, mask=m)
            bkt = ((sv >> (shift - 8)) & 0xFF) + laneoff
            plsc.addupdate_scatter(hist, [bkt], ones, mask=m)
            return off + plsc.all_reduce_population_count(m)

        return jnp.max(off)

    def row_body(rr, _):
        row = wid * ROWS_PER_W + rr
        pltpu.sync_copy(x_hbm.at[pl.ds(row * N, N)], xv)

        clear_hist()

        @plsc.parallel_loop(0, NV, unroll=8)
        def _(i):
            xf = xv[pl.ds(i * L, L)]
            b = lax.bitcast_convert_type(xf, jnp.int32)
            s = jnp.where(b < 0, b ^ jnp.int32(0x7FFFFFFF), b)
            bkt = (s >> 24) + 128 + laneoff
            plsc.addupdate_scatter(hist, [bkt], ones, mask=true16)

        b1, r = search(jnp.int32(K))
        p1 = b1 - 128

        @plsc.parallel_loop(0, NV, unroll=8, carry=zero16)
        def off1(i, off):
            xf = xv[pl.ds(i * L, L)]
            b = lax.bitcast_convert_type(xf, jnp.int32)
            s = jnp.where(b < 0, b ^ jnp.int32(0x7FFFFFFF), b)
            m = (s >> 24) == p1
            mi = jnp.where(m, ones, zero16)
            pos = off + plsc.cumsum(mi) - 1
            plsc.store_scatter(ca, [pos], s, mask=m)
            return off + plsc.all_reduce_population_count(m)

        n1 = jnp.max(off1)

        clear_hist()
        nv1 = (n1 + L - 1) // L

        @plsc.parallel_loop(0, nv1, unroll=8)
        def _(i):
            sv = ca[pl.ds(i * L, L)]
            inb = (i * L + lanes) < n1
            bkt = ((sv >> 16) & 0xFF) + laneoff
            plsc.addupdate_scatter(hist, [bkt], ones, mask=inb)

        b2, r = search(r)
        p2 = (p1 << 8) | b2
        n2 = compact(ca, cb, n1, 16, p2)

        b3, r = search(r)
        p3 = (p2 << 8) | b3
        n3 = compact(cb, ca, n2, 8, p3)

        b4, r = search(r)
        thr = (p3 << 8) | b4

        tv = zero16 + thr
        bv = jnp.where(tv >= 0, tv, tv ^ jnp.int32(0x7FFFFFFF))
        fv = lax.bitcast_convert_type(bv, jnp.float32)
        for j in range(TW // L):
            tbuf[pl.ds(rr * TW + j * L, L)] = fv
        return 0

    lax.fori_loop(0, ROWS_PER_W, row_body, 0)
    pltpu.sync_copy(tbuf, thr_hbm.at[pl.ds(wid * (ROWS_PER_W * TW),
                                           ROWS_PER_W * TW)])


def _mask_body(thr_ref, x_ref, o_ref):
    t = thr_ref[:, 0:1]
    xb = x_ref[...]
    o_ref[...] = jnp.where(xb >= t, xb, 0.0)


_mask = pl.pallas_call(
    _mask_body,
    grid=(16,),
    in_specs=[
        pl.BlockSpec((8, TW), lambda i: (i, 0)),
        pl.BlockSpec((8, N), lambda i: (i, 0)),
    ],
    out_specs=pl.BlockSpec((8, N), lambda i: (i, 0)),
    out_shape=jax.ShapeDtypeStruct((B, N), jnp.float32),
)


def kernel(x):
    thr = _select_thr(x.reshape(B * N)).reshape(B, TW)
    return _mask(thr, x)


# speedup vs baseline: 1.0641x; 1.0641x over previous
import functools
import math

import jax
import jax.numpy as jnp
from jax import lax
from jax.experimental import pallas as pl
from jax.experimental.pallas import tpu as pltpu
from jax.experimental.pallas import tpu_sc as plsc

B = 128
N = 32768
K = math.ceil(0.05 * N)

NC = 2
NS = 16
L = 16
NW = NC * NS
ROWS_PER_W = B // NW
NV = N // L
NBKT = 256
HSTRIDE = 257
HWORDS = 264 * L
TW = 128


def _scal(v):
    if getattr(v, "ndim", 0) == 1:
        return jnp.max(v)
    return v


_mesh = plsc.VectorSubcoreMesh(core_axis_name="c", subcore_axis_name="s")


@functools.partial(
    pl.kernel,
    mesh=_mesh,
    out_type=jax.ShapeDtypeStruct((B * TW,), jnp.float32),
    compiler_params=pltpu.CompilerParams(needs_layout_passes=False),
    scratch_types=[
        pltpu.VMEM((N,), jnp.float32),
        pltpu.VMEM((N,), jnp.float32),
        pltpu.VMEM((N,), jnp.int32),
        pltpu.VMEM((HWORDS,), jnp.int32),
        pltpu.VMEM((NBKT,), jnp.int32),
        pltpu.VMEM((ROWS_PER_W * TW,), jnp.float32),
        pltpu.SemaphoreType.DMA,
        pltpu.SemaphoreType.DMA,
    ],
)
def _select_thr(x_hbm, thr_hbm, xv0, xv1, ca, hist, merged, tbuf, sema, semb):
    wid = lax.axis_index("s") * NC + lax.axis_index("c")
    lanes = lax.iota(jnp.int32, L)
    laneoff = lanes * HSTRIDE
    ones = jnp.ones((L,), jnp.int32)
    zero16 = jnp.zeros((L,), jnp.int32)
    true16 = lanes >= 0
    row0 = wid * ROWS_PER_W

    def clear_hist():
        @plsc.parallel_loop(0, HWORDS // L, unroll=8)
        def _(i):
            hist[pl.ds(i * L, L)] = zero16

    def search(r0):
        def ga_body(t, carry):
            r, found, grp, rg = carry
            g = 15 - t
            acc = zero16
            for l in range(L):
                acc = acc + hist[pl.ds(l * HSTRIDE + g * L, L)]
            merged[pl.ds(g * L, L)] = acc
            tot = jnp.sum(acc)
            hit = jnp.logical_and(found == 0, tot >= r)
            grp = jnp.where(hit, g, grp)
            rg = jnp.where(hit, r, rg)
            r = jnp.where(jnp.logical_and(found == 0, jnp.logical_not(hit)),
                          r - tot, r)
            found = jnp.where(hit, jnp.int32(1), found)
            return (r, found, grp, rg)

        r, _found, grp, rg = lax.fori_loop(
            0, 16, ga_body,
            (r0, jnp.int32(0), jnp.int32(0), jnp.int32(1)))

        acc = merged[pl.ds(grp * L, L)]
        rev = lax.rev(acc, (0,))
        c = plsc.cumsum(rev)
        mge = c >= rg
        i0 = _scal(plsc.all_reduce_ffs(mge))
        msel = lanes == i0
        ci0 = jnp.sum(jnp.where(msel, c, zero16))
        ri0 = jnp.sum(jnp.where(msel, rev, zero16))
        bucket = grp * L + (15 - i0)
        r_next = rg - (ci0 - ri0)
        return bucket, r_next

    def histo(n_src, shift):
        clear_hist()
        nv = (n_src + L - 1) // L

        @plsc.parallel_loop(0, nv, unroll=8)
        def _(i):
            sv = ca[pl.ds(i * L, L)]
            inb = (i * L + lanes) < n_src
            bkt = ((sv >> shift) & 0xFF) + laneoff
            plsc.addupdate_scatter(hist, [bkt], ones, mask=inb)

    def compact(n_src, shift, p):
        nv = (n_src + L - 1) // L

        @plsc.parallel_loop(0, nv, unroll=8, carry=zero16)
        def off(i, off):
            sv = ca[pl.ds(i * L, L)]
            inb = (i * L + lanes) < n_src
            m = jnp.logical_and((sv >> shift) == p, inb)
            mi = jnp.where(m, ones, zero16)
            pos = off + plsc.cumsum(mi) - 1
            plsc.store_scatter(ca, [pos], sv, mask=m)
            return off + plsc.all_reduce_population_count(m)

        return jnp.max(off)

    def process(xv, rr):
        clear_hist()

        @plsc.parallel_loop(0, NV, unroll=8)
        def _(i):
            xf = xv[pl.ds(i * L, L)]
            b = lax.bitcast_convert_type(xf, jnp.int32)
            s = jnp.where(b < 0, b ^ jnp.int32(0x7FFFFFFF), b)
            bkt = (s >> 24) + 128 + laneoff
            plsc.addupdate_scatter(hist, [bkt], ones, mask=true16)

        b1, r = search(jnp.int32(K))
        p1 = b1 - 128

        @plsc.parallel_loop(0, NV, unroll=8, carry=zero16)
        def off1(i, off):
            xf = xv[pl.ds(i * L, L)]
            b = lax.bitcast_convert_type(xf, jnp.int32)
            s = jnp.where(b < 0, b ^ jnp.int32(0x7FFFFFFF), b)
            m = (s >> 24) == p1
            mi = jnp.where(m, ones, zero16)
            pos = off + plsc.cumsum(mi) - 1
            plsc.store_scatter(ca, [pos], s, mask=m)
            return off + plsc.all_reduce_population_count(m)

        n1 = jnp.max(off1)

        histo(n1, 16)
        b2, r = search(r)
        p2 = (p1 << 8) | b2
        n2 = compact(n1, 16, p2)

        histo(n2, 8)
        b3, r = search(r)
        p3 = (p2 << 8) | b3
        n3 = compact(n2, 8, p3)

        histo(n3, 0)
        b4, r = search(r)
        thr = (p3 << 8) | b4

        tv = zero16 + thr
        bv = jnp.where(tv >= 0, tv, tv ^ jnp.int32(0x7FFFFFFF))
        fv = lax.bitcast_convert_type(bv, jnp.float32)
        for j in range(TW // L):
            tbuf[pl.ds(rr * TW + j * L, L)] = fv

    pltpu.async_copy(x_hbm.at[pl.ds(row0 * N, N)], xv0, sema)

    def pair_body(pp, _):
        r_even = row0 + 2 * pp
        pltpu.make_async_copy(
            x_hbm.at[pl.ds(r_even * N, N)], xv0, sema).wait()
        pltpu.async_copy(x_hbm.at[pl.ds((r_even + 1) * N, N)], xv1, semb)
        process(xv0, 2 * pp)
        pltpu.make_async_copy(
            x_hbm.at[pl.ds((r_even + 1) * N, N)], xv1, semb).wait()

        @pl.when(pp == 0)
        def _():
            pltpu.async_copy(
                x_hbm.at[pl.ds((r_even + 2) * N, N)], xv0, sema)

        process(xv1, 2 * pp + 1)
        return 0

    lax.fori_loop(0, ROWS_PER_W // 2, pair_body, 0)
    pltpu.sync_copy(tbuf, thr_hbm.at[pl.ds(wid * (ROWS_PER_W * TW),
                                           ROWS_PER_W * TW)])


def _mask_body(thr_ref, x_ref, o_ref):
    t = thr_ref[:, 0:1]
    xb = x_ref[...]
    o_ref[...] = jnp.where(xb >= t, xb, 0.0)


_mask = pl.pallas_call(
    _mask_body,
    grid=(16,),
    in_specs=[
        pl.BlockSpec((8, TW), lambda i: (i, 0)),
        pl.BlockSpec((8, N), lambda i: (i, 0)),
    ],
    out_specs=pl.BlockSpec((8, N), lambda i: (i, 0)),
    out_shape=jax.ShapeDtypeStruct((B, N), jnp.float32),
)


def kernel(x):
    thr = _select_thr(x.reshape(B * N)).reshape(B, TW)
    return _mask(thr, x)
